# baseline (speedup 1.0000x reference)
import functools

import jax
import jax.numpy as jnp
from jax import lax
from jax.experimental import pallas as pl
from jax.experimental.pallas import tpu as pltpu
from jax.experimental.pallas import tpu_sc as plsc

NC = 2
NS = 16
NW = NC * NS

CHUNK = 256
SUB = CHUNK // 128
NCOPY = 4


def _sc_partial_sums(x, batch2d, n_rows, d, g):
    num_chunks = n_rows // CHUNK
    k_max = -(-num_chunks // NW)
    unif = k_max - 1
    tail_n = num_chunks - NW * unif
    mesh = plsc.VectorSubcoreMesh(
        core_axis_name="c", subcore_axis_name="s", num_cores=NC, num_subcores=NS
    )
    rows_per_tile = g // NS
    zrows = 8

    @functools.partial(
        pl.kernel,
        out_type=jax.ShapeDtypeStruct((NC * NCOPY, g, d), jnp.float32),
        mesh=mesh,
        scratch_types=[
            pltpu.VMEM((2, SUB, 128), jnp.int32),
            pltpu.VMEM((2, CHUNK, d), jnp.float32),
            pltpu.VMEM((zrows, d), jnp.float32),
            pltpu.VMEM_SHARED((NCOPY, g, d), jnp.float32),
            pltpu.SemaphoreType.DMA,
            pltpu.SemaphoreType.DMA,
        ],
    )
    def sc_kernel(x_hbm, b_hbm, out_hbm, ids_v, rows_v, zbuf_v, acc, sem0, sem1):
        cid = lax.axis_index("c")
        sid = lax.axis_index("s")
        wid = sid * NC + cid
        my_acc = acc.at[sid % NCOPY]
        sems = (sem0, sem1)

        zeros16 = jnp.zeros((16,), jnp.float32)

        @pl.loop(0, zrows)
        def _zero(i):
            for k in range(d // 16):
                zbuf_v[i, pl.ds(k * 16, 16)] = zeros16

        for c in range(NCOPY):
            for i in range(rows_per_tile // zrows):
                pltpu.sync_copy(
                    zbuf_v,
                    acc.at[c, pl.ds(sid * rows_per_tile + i * zrows, zrows)],
                )
        plsc.subcore_barrier()

        def start(k):
            j = wid + NW * k
            b = k % 2
            dr = pltpu.async_copy(
                x_hbm.at[pl.ds(j * CHUNK, CHUNK)], rows_v.at[b], sems[b]
            )
            di = pltpu.async_copy(
                b_hbm.at[pl.ds(j * SUB, SUB)], ids_v.at[b], sems[b]
            )
            return dr, di

        def scatter(b):
            for s in range(SUB):
                pltpu.sync_copy(
                    rows_v.at[b, pl.ds(s * 128, 128)],
                    my_acc.at[ids_v.at[b, s]],
                    add=True,
                )

        descs = {0: start(0)}
        for k in range(unif):
            if k + 1 < unif:
                descs[k + 1] = start(k + 1)
            dr, di = descs.pop(k)
            dr.wait()
            di.wait()
            scatter(k % 2)

        @pl.when(wid < tail_n)
        def _tail():
            j = wid + NW * unif
            pltpu.sync_copy(x_hbm.at[pl.ds(j * CHUNK, CHUNK)], rows_v.at[0])
            pltpu.sync_copy(b_hbm.at[pl.ds(j * SUB, SUB)], ids_v.at[0])
            scatter(0)

        plsc.subcore_barrier()
        for c in range(NCOPY):
            pltpu.sync_copy(
                acc.at[c, pl.ds(sid * rows_per_tile, rows_per_tile)],
                out_hbm.at[cid * NCOPY + c, pl.ds(sid * rows_per_tile, rows_per_tile)],
            )

    return sc_kernel(x, batch2d)


def _combine_body(p_ref, o_ref):
    s = p_ref[0]
    for c in range(1, NC * NCOPY):
        s = s + p_ref[c]
    o_ref[...] = s


def kernel(x, batch, ptr):
    n, d = x.shape
    g = int(ptr.shape[0]) - 1
    batch2d = batch.astype(jnp.int32).reshape(n // 128, 128)
    partials = _sc_partial_sums(x, batch2d, n, d, g)
    out = pl.pallas_call(
        _combine_body,
        out_shape=jax.ShapeDtypeStruct((g, d), jnp.float32),
    )(partials)
    return out


# speedup vs baseline: 1.0544x; 1.0544x over previous
import functools

import jax
import jax.numpy as jnp
from jax import lax
from jax.experimental import pallas as pl
from jax.experimental.pallas import tpu as pltpu
from jax.experimental.pallas import tpu_sc as plsc

NC = 2
NS = 16
NW = NC * NS

CHUNK = 256
SUB = CHUNK // 128


def _sc_partial_sums(x, batch2d, n_rows, d, g):
    num_chunks = n_rows // CHUNK
    k_max = -(-num_chunks // NW)
    unif = k_max - 1
    tail_n = num_chunks - NW * unif
    mesh = plsc.VectorSubcoreMesh(
        core_axis_name="c", subcore_axis_name="s", num_cores=NC, num_subcores=NS
    )
    rows_per_tile = g // NS
    zrows = 8

    @functools.partial(
        pl.kernel,
        out_type=jax.ShapeDtypeStruct((NC, g, d), jnp.float32),
        mesh=mesh,
        scratch_types=[
            pltpu.VMEM((2, SUB, 128), jnp.int32),
            pltpu.VMEM((2, CHUNK, d), jnp.float32),
            pltpu.VMEM((zrows, d), jnp.float32),
            pltpu.VMEM_SHARED((g, d), jnp.float32),
            pltpu.SemaphoreType.DMA,
            pltpu.SemaphoreType.DMA,
            pltpu.SemaphoreType.DMA,
            pltpu.SemaphoreType.DMA,
        ],
    )
    def sc_kernel(
        x_hbm, b_hbm, out_hbm, ids_v, rows_v, zbuf_v, acc, sem0, sem1, ssem0, ssem1
    ):
        cid = lax.axis_index("c")
        sid = lax.axis_index("s")
        wid = sid * NC + cid
        sems = (sem0, sem1)
        ssems = (ssem0, ssem1)

        zeros16 = jnp.zeros((16,), jnp.float32)

        @pl.loop(0, zrows)
        def _zero(i):
            for k in range(d // 16):
                zbuf_v[i, pl.ds(k * 16, 16)] = zeros16

        for i in range(rows_per_tile // zrows):
            pltpu.sync_copy(
                zbuf_v, acc.at[pl.ds(sid * rows_per_tile + i * zrows, zrows)]
            )
        plsc.subcore_barrier()

        def start(k):
            j = wid + NW * k
            b = k % 2
            dr = pltpu.async_copy(
                x_hbm.at[pl.ds(j * CHUNK, CHUNK)], rows_v.at[b], sems[b]
            )
            di = pltpu.async_copy(
                b_hbm.at[pl.ds(j * SUB, SUB)], ids_v.at[b], sems[b]
            )
            return dr, di

        def scatter(b):
            return [
                pltpu.async_copy(
                    rows_v.at[b, pl.ds(s * 128, 128)],
                    acc.at[ids_v.at[b, s]],
                    ssems[b],
                    add=True,
                )
                for s in range(SUB)
            ]

        sdescs = {0: [], 1: []}
        descs = {0: start(0)}
        for k in range(unif):
            b = k % 2
            if k + 1 < unif:
                nxt = 1 - b
                for sd in sdescs[nxt]:
                    sd.wait()
                descs[k + 1] = start(k + 1)
            dr, di = descs.pop(k)
            dr.wait()
            di.wait()
            sdescs[b] = scatter(b)

        for b in (0, 1):
            for sd in sdescs[b]:
                sd.wait()

        @pl.when(wid < tail_n)
        def _tail():
            j = wid + NW * unif
            pltpu.sync_copy(x_hbm.at[pl.ds(j * CHUNK, CHUNK)], rows_v.at[0])
            pltpu.sync_copy(b_hbm.at[pl.ds(j * SUB, SUB)], ids_v.at[0])
            for s in range(SUB):
                pltpu.sync_copy(
                    rows_v.at[0, pl.ds(s * 128, 128)],
                    acc.at[ids_v.at[0, s]],
                    add=True,
                )

        plsc.subcore_barrier()
        pltpu.sync_copy(
            acc.at[pl.ds(sid * rows_per_tile, rows_per_tile)],
            out_hbm.at[cid, pl.ds(sid * rows_per_tile, rows_per_tile)],
        )

    return sc_kernel(x, batch2d)


def _combine_body(p_ref, o_ref):
    o_ref[...] = p_ref[0] + p_ref[1]


def kernel(x, batch, ptr):
    n, d = x.shape
    g = int(ptr.shape[0]) - 1
    batch2d = batch.astype(jnp.int32).reshape(n // 128, 128)
    partials = _sc_partial_sums(x, batch2d, n, d, g)
    out = pl.pallas_call(
        _combine_body,
        out_shape=jax.ShapeDtypeStruct((g, d), jnp.float32),
    )(partials)
    return out
